# baseline (speedup 1.0000x reference)
import functools

import jax
import jax.numpy as jnp
from jax import lax
from jax.experimental import pallas as pl
from jax.experimental.pallas import tpu as pltpu
from jax.experimental.pallas import tpu_sc as plsc

NC = 2
NS = 16
NW = NC * NS
CHUNK = 1000
DEGW = 16


def _mesh():
    return plsc.VectorSubcoreMesh(core_axis_name="c", subcore_axis_name="s")


def _deg_partials(dst, ones_hbm, zeros_hbm, n, chunk):
    e = dst.shape[0]
    nch = e // (NW * chunk)
    dst3 = dst.reshape(NW, nch, chunk)
    rows_per_out = n // 10

    @functools.partial(
        pl.kernel,
        out_type=jax.ShapeDtypeStruct((NC, n, DEGW), jnp.float32),
        mesh=_mesh(),
        scratch_types=[
            pltpu.VMEM((nch, chunk), jnp.int32),
            pltpu.VMEM((chunk, DEGW), jnp.float32),
            pltpu.VMEM_SHARED((n, DEGW), jnp.float32),
        ],
        compiler_params=pltpu.CompilerParams(use_tc_tiling_on_sc=False),
    )
    def deg_k(dst_hbm, ones_h, zeros_h, out_hbm, idx_d, ones_v, acc):
        ci = lax.axis_index("c")
        s = lax.axis_index("s")
        wid = ci * NS + s
        pltpu.sync_copy(dst_hbm.at[wid], idx_d)
        pltpu.sync_copy(ones_h, ones_v)

        @pl.when(s < 10)
        def _zero():
            sl = pl.ds(pl.multiple_of(s * rows_per_out, 8), rows_per_out)
            pltpu.sync_copy(zeros_h, acc.at[sl])

        plsc.subcore_barrier()

        def body(j, carry):
            pltpu.sync_copy(ones_v, acc.at[idx_d.at[j]], add=True)
            return carry

        lax.fori_loop(0, nch, body, 0)
        plsc.subcore_barrier()

        @pl.when(s < 10)
        def _out():
            sl = pl.ds(pl.multiple_of(s * rows_per_out, 8), rows_per_out)
            pltpu.sync_copy(acc.at[sl], out_hbm.at[ci].at[sl])

    return deg_k(dst3, ones_hbm, zeros_hbm)


def _agg_partials(y, src, dst, zeros_hbm, n, d, chunk):
    e = src.shape[0]
    nch = e // (NW * chunk)
    src3 = src.reshape(NW, nch, chunk)
    dst3 = dst.reshape(NW, nch, chunk)
    rows_per_out = n // 10

    @functools.partial(
        pl.kernel,
        out_type=jax.ShapeDtypeStruct((NC, n, d), jnp.float32),
        mesh=_mesh(),
        scratch_types=[
            pltpu.VMEM((nch, chunk), jnp.int32),
            pltpu.VMEM((nch, chunk), jnp.int32),
            pltpu.VMEM((chunk, d), jnp.float32),
            pltpu.VMEM((chunk, d), jnp.float32),
            pltpu.VMEM_SHARED((n, d), jnp.float32),
            pltpu.SemaphoreType.DMA,
            pltpu.SemaphoreType.DMA,
            pltpu.SemaphoreType.DMA,
            pltpu.SemaphoreType.DMA,
        ],
        compiler_params=pltpu.CompilerParams(use_tc_tiling_on_sc=False),
    )
    def agg_k(y_hbm, src_hbm, dst_hbm, zeros_h, out_hbm,
              idx_s, idx_d, rows_a, rows_b, acc, ga, gb, sa, sb):
        ci = lax.axis_index("c")
        s = lax.axis_index("s")
        wid = ci * NS + s
        pltpu.sync_copy(src_hbm.at[wid], idx_s)
        pltpu.sync_copy(dst_hbm.at[wid], idx_d)

        @pl.when(s < 10)
        def _zero():
            sl = pl.ds(pl.multiple_of(s * rows_per_out, 8), rows_per_out)
            pltpu.sync_copy(zeros_h, acc.at[sl])

        plsc.subcore_barrier()

        def gather(j, buf, sem):
            return pltpu.async_copy(y_hbm.at[idx_s.at[j]], buf, sem)

        def scat(j, buf, sem):
            return pltpu.async_copy(buf, acc.at[idx_d.at[j]], sem, add=True)

        h_ga = gather(0, rows_a, ga)
        h_sb = None
        for i in range(nch // 2):
            h_ga.wait()
            if h_sb is not None:
                h_sb.wait()
            h_gb = gather(2 * i + 1, rows_b, gb)
            h_sa = scat(2 * i, rows_a, sa)
            h_gb.wait()
            h_sa.wait()
            if i + 1 < nch // 2:
                h_ga = gather(2 * i + 2, rows_a, ga)
            h_sb = scat(2 * i + 1, rows_b, sb)
        h_sb.wait()
        plsc.subcore_barrier()

        @pl.when(s < 10)
        def _out():
            sl = pl.ds(pl.multiple_of(s * rows_per_out, 8), rows_per_out)
            pltpu.sync_copy(acc.at[sl], out_hbm.at[ci].at[sl])

    return agg_k(y, src3, dst3, zeros_hbm)


def _mm_body(x_ref, w_ref, o_ref):
    o_ref[...] = jnp.dot(x_ref[...], w_ref[...], preferred_element_type=jnp.float32)


def _scale_body(z_ref, dp_ref, y_ref, dinv_ref):
    deg = dp_ref[0, :, 0:1] + dp_ref[1, :, 0:1] + 1.0
    dinv = lax.rsqrt(deg)
    dinv_ref[...] = dinv
    y_ref[...] = z_ref[...] * dinv


def _mid_body(p_ref, y1_ref, dinv_ref, b1_ref, w2_ref, y2_ref):
    agg = p_ref[0] + p_ref[1] + y1_ref[...]
    h = jnp.maximum(agg * dinv_ref[...] + b1_ref[...], 0.0)
    z2 = jnp.dot(h, w2_ref[...], preferred_element_type=jnp.float32)
    y2_ref[...] = z2 * dinv_ref[...]


def _fin_body(q_ref, y2_ref, dinv_ref, b2_ref, o_ref):
    o_ref[...] = (q_ref[0] + q_ref[1] + y2_ref[...]) * dinv_ref[...] + b2_ref[...]


def _sds(shape):
    return jax.ShapeDtypeStruct(shape, jnp.float32)


def kernel(x, edge_index, W1, b1, W2, b2):
    n, in_dim = x.shape
    hid = W1.shape[1]
    out_dim = W2.shape[1]
    p2 = 16
    src = edge_index[0].astype(jnp.int32)
    dst = edge_index[1].astype(jnp.int32)
    ones1 = jnp.ones((1000, DEGW), jnp.float32)
    zeros1 = jnp.zeros((n // 10, DEGW), jnp.float32)
    zeros_h = jnp.zeros((n // 10, hid), jnp.float32)
    zeros_p = jnp.zeros((n // 10, p2), jnp.float32)
    W2p = jnp.zeros((hid, p2), jnp.float32).at[:, :out_dim].set(W2)
    b1r = b1.reshape(1, hid)
    b2p = jnp.zeros((1, p2), jnp.float32).at[0, :out_dim].set(b2)

    bs = 2000
    g = n // bs

    z1 = pl.pallas_call(
        _mm_body,
        grid=(g,),
        in_specs=[pl.BlockSpec((bs, in_dim), lambda i: (i, 0)),
                  pl.BlockSpec((in_dim, hid), lambda i: (0, 0))],
        out_specs=pl.BlockSpec((bs, hid), lambda i: (i, 0)),
        out_shape=_sds((n, hid)))(x, W1)
    degp = _deg_partials(dst, ones1, zeros1, n, 1000)

    y1, dinv = pl.pallas_call(
        _scale_body,
        grid=(g,),
        in_specs=[pl.BlockSpec((bs, hid), lambda i: (i, 0)),
                  pl.BlockSpec((NC, bs, DEGW), lambda i: (0, i, 0))],
        out_specs=(pl.BlockSpec((bs, hid), lambda i: (i, 0)),
                   pl.BlockSpec((bs, 1), lambda i: (i, 0))),
        out_shape=(_sds((n, hid)), _sds((n, 1))))(z1, degp)

    p1 = _agg_partials(y1, src, dst, zeros_h, n, hid, 500)
    y2 = pl.pallas_call(
        _mid_body,
        grid=(g,),
        in_specs=[pl.BlockSpec((NC, bs, hid), lambda i: (0, i, 0)),
                  pl.BlockSpec((bs, hid), lambda i: (i, 0)),
                  pl.BlockSpec((bs, 1), lambda i: (i, 0)),
                  pl.BlockSpec((1, hid), lambda i: (0, 0)),
                  pl.BlockSpec((hid, p2), lambda i: (0, 0))],
        out_specs=pl.BlockSpec((bs, p2), lambda i: (i, 0)),
        out_shape=_sds((n, p2)))(p1, y1, dinv, b1r, W2p)

    q1 = _agg_partials(y2, src, dst, zeros_p, n, p2, 1000)
    out16 = pl.pallas_call(
        _fin_body,
        grid=(g,),
        in_specs=[pl.BlockSpec((NC, bs, p2), lambda i: (0, i, 0)),
                  pl.BlockSpec((bs, p2), lambda i: (i, 0)),
                  pl.BlockSpec((bs, 1), lambda i: (i, 0)),
                  pl.BlockSpec((1, p2), lambda i: (0, 0))],
        out_specs=pl.BlockSpec((bs, p2), lambda i: (i, 0)),
        out_shape=_sds((n, p2)))(q1, y2, dinv, b2p)
    return out16[:, :out_dim]


# speedup vs baseline: 1.0126x; 1.0126x over previous
import functools

import jax
import jax.numpy as jnp
from jax import lax
from jax.experimental import pallas as pl
from jax.experimental.pallas import tpu as pltpu
from jax.experimental.pallas import tpu_sc as plsc

NC = 2
NS = 16
NW = NC * NS
CHUNK = 1000
DEGW = 16


def _mesh():
    return plsc.VectorSubcoreMesh(core_axis_name="c", subcore_axis_name="s")


def _deg_partials(dst, ones_hbm, zeros_hbm, n, chunk):
    e = dst.shape[0]
    nch = e // (NW * chunk)
    dst3 = dst.reshape(NW, nch, chunk)
    rows_per_out = n // 10

    @functools.partial(
        pl.kernel,
        out_type=jax.ShapeDtypeStruct((NC, n, DEGW), jnp.float32),
        mesh=_mesh(),
        scratch_types=[
            pltpu.VMEM((nch, chunk), jnp.int32),
            pltpu.VMEM((chunk, DEGW), jnp.float32),
            pltpu.VMEM_SHARED((n, DEGW), jnp.float32),
        ],
        compiler_params=pltpu.CompilerParams(use_tc_tiling_on_sc=False),
    )
    def deg_k(dst_hbm, ones_h, zeros_h, out_hbm, idx_d, ones_v, acc):
        ci = lax.axis_index("c")
        s = lax.axis_index("s")
        wid = ci * NS + s
        pltpu.sync_copy(dst_hbm.at[wid], idx_d)
        pltpu.sync_copy(ones_h, ones_v)

        @pl.when(s < 10)
        def _zero():
            sl = pl.ds(pl.multiple_of(s * rows_per_out, 8), rows_per_out)
            pltpu.sync_copy(zeros_h, acc.at[sl])

        plsc.subcore_barrier()

        def body(j, carry):
            pltpu.sync_copy(ones_v, acc.at[idx_d.at[j]], add=True)
            return carry

        lax.fori_loop(0, nch, body, 0)
        plsc.subcore_barrier()

        @pl.when(s < 10)
        def _out():
            sl = pl.ds(pl.multiple_of(s * rows_per_out, 8), rows_per_out)
            pltpu.sync_copy(acc.at[sl], out_hbm.at[ci].at[sl])

    return deg_k(dst3, ones_hbm, zeros_hbm)


def _agg_partials(y, src, dst, zeros_hbm, n, d, chunk):
    e = src.shape[0]
    nch = e // (NW * chunk)
    src3 = src.reshape(NW, nch, chunk)
    dst3 = dst.reshape(NW, nch, chunk)
    rows_per_out = n // 10

    @functools.partial(
        pl.kernel,
        out_type=jax.ShapeDtypeStruct((NC, n, d), jnp.float32),
        mesh=_mesh(),
        scratch_types=[
            pltpu.VMEM((nch, chunk), jnp.int32),
            pltpu.VMEM((nch, chunk), jnp.int32),
            pltpu.VMEM((chunk, d), jnp.float32),
            pltpu.VMEM((chunk, d), jnp.float32),
            pltpu.VMEM_SHARED((n, d), jnp.float32),
            pltpu.SemaphoreType.DMA,
            pltpu.SemaphoreType.DMA,
            pltpu.SemaphoreType.DMA,
            pltpu.SemaphoreType.DMA,
        ],
        compiler_params=pltpu.CompilerParams(use_tc_tiling_on_sc=False),
    )
    def agg_k(y_hbm, src_hbm, dst_hbm, zeros_h, out_hbm,
              idx_s, idx_d, rows_a, rows_b, acc, ga, gb, sa, sb):
        ci = lax.axis_index("c")
        s = lax.axis_index("s")
        wid = ci * NS + s
        pltpu.sync_copy(src_hbm.at[wid], idx_s)
        pltpu.sync_copy(dst_hbm.at[wid], idx_d)

        @pl.when(s < 10)
        def _zero():
            sl = pl.ds(pl.multiple_of(s * rows_per_out, 8), rows_per_out)
            pltpu.sync_copy(zeros_h, acc.at[sl])

        plsc.subcore_barrier()

        def gather(j, buf, sem):
            return pltpu.async_copy(y_hbm.at[idx_s.at[j]], buf, sem)

        def scat(j, buf, sem):
            return pltpu.async_copy(buf, acc.at[idx_d.at[j]], sem, add=True)

        h_ga = gather(0, rows_a, ga)
        h_sb = None
        for i in range(nch // 2):
            h_ga.wait()
            if h_sb is not None:
                h_sb.wait()
            h_gb = gather(2 * i + 1, rows_b, gb)
            h_sa = scat(2 * i, rows_a, sa)
            h_gb.wait()
            h_sa.wait()
            if i + 1 < nch // 2:
                h_ga = gather(2 * i + 2, rows_a, ga)
            h_sb = scat(2 * i + 1, rows_b, sb)
        h_sb.wait()
        plsc.subcore_barrier()

        @pl.when(s < 10)
        def _out():
            sl = pl.ds(pl.multiple_of(s * rows_per_out, 8), rows_per_out)
            pltpu.sync_copy(acc.at[sl], out_hbm.at[ci].at[sl])

    return agg_k(y, src3, dst3, zeros_hbm)


def _scale_body(x_ref, w1_ref, dp_ref, y_ref, dinv_ref):
    deg = dp_ref[0, :, 0:1] + dp_ref[1, :, 0:1] + 1.0
    dinv = lax.rsqrt(deg)
    dinv_ref[...] = dinv
    z = jnp.dot(x_ref[...], w1_ref[...], preferred_element_type=jnp.float32)
    y_ref[...] = z * dinv


def _mid_body(p_ref, y1_ref, dinv_ref, b1_ref, w2_ref, y2_ref):
    agg = p_ref[0] + p_ref[1] + y1_ref[...]
    h = jnp.maximum(agg * dinv_ref[...] + b1_ref[...], 0.0)
    z2 = jnp.dot(h, w2_ref[...], preferred_element_type=jnp.float32)
    y2_ref[...] = z2 * dinv_ref[...]


def _fin_body(q_ref, y2_ref, dinv_ref, b2_ref, o_ref):
    o_ref[...] = (q_ref[0] + q_ref[1] + y2_ref[...]) * dinv_ref[...] + b2_ref[...]


def _sds(shape):
    return jax.ShapeDtypeStruct(shape, jnp.float32)


def kernel(x, edge_index, W1, b1, W2, b2):
    n, in_dim = x.shape
    hid = W1.shape[1]
    out_dim = W2.shape[1]
    p2 = 16
    src = edge_index[0].astype(jnp.int32)
    dst = edge_index[1].astype(jnp.int32)
    ones1 = jnp.ones((1000, DEGW), jnp.float32)
    zeros1 = jnp.zeros((n // 10, DEGW), jnp.float32)
    zeros_h = jnp.zeros((n // 10, hid), jnp.float32)
    zeros_p = jnp.zeros((n // 10, p2), jnp.float32)
    W2p = jnp.zeros((hid, p2), jnp.float32).at[:, :out_dim].set(W2)
    b1r = b1.reshape(1, hid)
    b2p = jnp.zeros((1, p2), jnp.float32).at[0, :out_dim].set(b2)

    bs = 2000
    g = n // bs

    degp = _deg_partials(dst, ones1, zeros1, n, 1000)

    y1, dinv = pl.pallas_call(
        _scale_body,
        grid=(g,),
        in_specs=[pl.BlockSpec((bs, in_dim), lambda i: (i, 0)),
                  pl.BlockSpec((in_dim, hid), lambda i: (0, 0)),
                  pl.BlockSpec((NC, bs, DEGW), lambda i: (0, i, 0))],
        out_specs=(pl.BlockSpec((bs, hid), lambda i: (i, 0)),
                   pl.BlockSpec((bs, 1), lambda i: (i, 0))),
        out_shape=(_sds((n, hid)), _sds((n, 1))))(x, W1, degp)

    p1 = _agg_partials(y1, src, dst, zeros_h, n, hid, 500)
    y2 = pl.pallas_call(
        _mid_body,
        grid=(g,),
        in_specs=[pl.BlockSpec((NC, bs, hid), lambda i: (0, i, 0)),
                  pl.BlockSpec((bs, hid), lambda i: (i, 0)),
                  pl.BlockSpec((bs, 1), lambda i: (i, 0)),
                  pl.BlockSpec((1, hid), lambda i: (0, 0)),
                  pl.BlockSpec((hid, p2), lambda i: (0, 0))],
        out_specs=pl.BlockSpec((bs, p2), lambda i: (i, 0)),
        out_shape=_sds((n, p2)))(p1, y1, dinv, b1r, W2p)

    q1 = _agg_partials(y2, src, dst, zeros_p, n, p2, 1000)
    out16 = pl.pallas_call(
        _fin_body,
        grid=(g,),
        in_specs=[pl.BlockSpec((NC, bs, p2), lambda i: (0, i, 0)),
                  pl.BlockSpec((bs, p2), lambda i: (i, 0)),
                  pl.BlockSpec((bs, 1), lambda i: (i, 0)),
                  pl.BlockSpec((1, p2), lambda i: (0, 0))],
        out_specs=pl.BlockSpec((bs, p2), lambda i: (i, 0)),
        out_shape=_sds((n, p2)))(q1, y2, dinv, b2p)
    return out16[:, :out_dim]


# speedup vs baseline: 1.0760x; 1.0626x over previous
import functools

import jax
import jax.numpy as jnp
from jax import lax
from jax.experimental import pallas as pl
from jax.experimental.pallas import tpu as pltpu
from jax.experimental.pallas import tpu_sc as plsc

NC = 2
NS = 16
NW = NC * NS
CHUNK = 1000
DEGW = 8


def _mesh():
    return plsc.VectorSubcoreMesh(core_axis_name="c", subcore_axis_name="s")


def _deg_partials(dst, ones_hbm, zeros_hbm, n, chunk):
    e = dst.shape[0]
    nch = e // (NW * chunk)
    dst3 = dst.reshape(NW, nch, chunk)
    rows_per_out = n // 10

    @functools.partial(
        pl.kernel,
        out_type=jax.ShapeDtypeStruct((NC, n, DEGW), jnp.float32),
        mesh=_mesh(),
        scratch_types=[
            pltpu.VMEM((nch, chunk), jnp.int32),
            pltpu.VMEM((chunk, DEGW), jnp.float32),
            pltpu.VMEM_SHARED((n, DEGW), jnp.float32),
        ],
        compiler_params=pltpu.CompilerParams(use_tc_tiling_on_sc=False),
    )
    def deg_k(dst_hbm, ones_h, zeros_h, out_hbm, idx_d, ones_v, acc):
        ci = lax.axis_index("c")
        s = lax.axis_index("s")
        wid = ci * NS + s
        pltpu.sync_copy(dst_hbm.at[wid], idx_d)
        pltpu.sync_copy(ones_h, ones_v)

        @pl.when(s < 10)
        def _zero():
            sl = pl.ds(pl.multiple_of(s * rows_per_out, 8), rows_per_out)
            pltpu.sync_copy(zeros_h, acc.at[sl])

        plsc.subcore_barrier()

        def body(j, carry):
            pltpu.sync_copy(ones_v, acc.at[idx_d.at[j]], add=True)
            return carry

        lax.fori_loop(0, nch, body, 0)
        plsc.subcore_barrier()

        @pl.when(s < 10)
        def _out():
            sl = pl.ds(pl.multiple_of(s * rows_per_out, 8), rows_per_out)
            pltpu.sync_copy(acc.at[sl], out_hbm.at[ci].at[sl])

    return deg_k(dst3, ones_hbm, zeros_hbm)


def _agg_partials(y, src, dst, zeros_hbm, n, d, chunk):
    e = src.shape[0]
    nch = e // (NW * chunk)
    src3 = src.reshape(NW, nch, chunk)
    dst3 = dst.reshape(NW, nch, chunk)
    rows_per_out = n // 10

    @functools.partial(
        pl.kernel,
        out_type=jax.ShapeDtypeStruct((NC, n, d), jnp.float32),
        mesh=_mesh(),
        scratch_types=[
            pltpu.VMEM((nch, chunk), jnp.int32),
            pltpu.VMEM((nch, chunk), jnp.int32),
            pltpu.VMEM((chunk, d), jnp.float32),
            pltpu.VMEM((chunk, d), jnp.float32),
            pltpu.VMEM_SHARED((n, d), jnp.float32),
            pltpu.SemaphoreType.DMA,
            pltpu.SemaphoreType.DMA,
            pltpu.SemaphoreType.DMA,
            pltpu.SemaphoreType.DMA,
        ],
        compiler_params=pltpu.CompilerParams(use_tc_tiling_on_sc=False),
    )
    def agg_k(y_hbm, src_hbm, dst_hbm, zeros_h, out_hbm,
              idx_s, idx_d, rows_a, rows_b, acc, ga, gb, sa, sb):
        ci = lax.axis_index("c")
        s = lax.axis_index("s")
        wid = ci * NS + s
        pltpu.sync_copy(src_hbm.at[wid], idx_s)
        pltpu.sync_copy(dst_hbm.at[wid], idx_d)

        @pl.when(s < 10)
        def _zero():
            sl = pl.ds(pl.multiple_of(s * rows_per_out, 8), rows_per_out)
            pltpu.sync_copy(zeros_h, acc.at[sl])

        plsc.subcore_barrier()

        def gather(j, buf, sem):
            return pltpu.async_copy(y_hbm.at[idx_s.at[j]], buf, sem)

        def scat(j, buf, sem):
            return pltpu.async_copy(buf, acc.at[idx_d.at[j]], sem, add=True)

        h_ga = gather(0, rows_a, ga)
        h_sb = None
        for i in range(nch // 2):
            h_ga.wait()
            if h_sb is not None:
                h_sb.wait()
            h_gb = gather(2 * i + 1, rows_b, gb)
            h_sa = scat(2 * i, rows_a, sa)
            h_gb.wait()
            h_sa.wait()
            if i + 1 < nch // 2:
                h_ga = gather(2 * i + 2, rows_a, ga)
            h_sb = scat(2 * i + 1, rows_b, sb)
        h_sb.wait()
        plsc.subcore_barrier()

        @pl.when(s < 10)
        def _out():
            sl = pl.ds(pl.multiple_of(s * rows_per_out, 8), rows_per_out)
            pltpu.sync_copy(acc.at[sl], out_hbm.at[ci].at[sl])

    return agg_k(y, src3, dst3, zeros_hbm)


def _scale_body(x_ref, w1_ref, dpw_ref, y_ref, dinv_ref):
    n = x_ref.shape[0]
    cnt = dpw_ref[0] + dpw_ref[1]
    cnt = cnt.reshape(n // 16, 16, DEGW)[:, :, 0]
    deg = cnt.reshape(n, 1) + 1.0
    dinv = lax.rsqrt(deg)
    dinv_ref[...] = dinv
    z = jnp.dot(x_ref[...], w1_ref[...], preferred_element_type=jnp.float32)
    y_ref[...] = z * dinv


def _mid_body(p_ref, y1_ref, dinv_ref, b1_ref, w2_ref, y2_ref):
    agg = p_ref[0] + p_ref[1] + y1_ref[...]
    h = jnp.maximum(agg * dinv_ref[...] + b1_ref[...], 0.0)
    z2 = jnp.dot(h, w2_ref[...], preferred_element_type=jnp.float32)
    y2_ref[...] = z2 * dinv_ref[...]


def _fin_body(q_ref, y2_ref, dinv_ref, b2_ref, o_ref):
    o_ref[...] = (q_ref[0] + q_ref[1] + y2_ref[...]) * dinv_ref[...] + b2_ref[...]


def _sds(shape):
    return jax.ShapeDtypeStruct(shape, jnp.float32)


def kernel(x, edge_index, W1, b1, W2, b2):
    n, in_dim = x.shape
    hid = W1.shape[1]
    out_dim = W2.shape[1]
    p2 = 16
    src = edge_index[0].astype(jnp.int32)
    dst = edge_index[1].astype(jnp.int32)
    ones1 = jnp.ones((1000, DEGW), jnp.float32)
    zeros1 = jnp.zeros((n // 10, DEGW), jnp.float32)
    zeros_h = jnp.zeros((n // 10, hid), jnp.float32)
    zeros_p = jnp.zeros((n // 10, p2), jnp.float32)
    W2p = jnp.zeros((hid, p2), jnp.float32).at[:, :out_dim].set(W2)
    b1r = b1.reshape(1, hid)
    b2p = jnp.zeros((1, p2), jnp.float32).at[0, :out_dim].set(b2)

    bs = 2000
    g = n // bs

    degp = _deg_partials(dst, ones1, zeros1, n, 1000)
    dpw = degp.reshape(NC, n // (128 // DEGW), 128)

    y1, dinv = pl.pallas_call(
        _scale_body,
        out_shape=(_sds((n, hid)), _sds((n, 1))))(x, W1, dpw)

    p1 = _agg_partials(y1, src, dst, zeros_h, n, hid, 500)
    y2 = pl.pallas_call(
        _mid_body,
        grid=(g,),
        in_specs=[pl.BlockSpec((NC, bs, hid), lambda i: (0, i, 0)),
                  pl.BlockSpec((bs, hid), lambda i: (i, 0)),
                  pl.BlockSpec((bs, 1), lambda i: (i, 0)),
                  pl.BlockSpec((1, hid), lambda i: (0, 0)),
                  pl.BlockSpec((hid, p2), lambda i: (0, 0))],
        out_specs=pl.BlockSpec((bs, p2), lambda i: (i, 0)),
        out_shape=_sds((n, p2)))(p1, y1, dinv, b1r, W2p)

    q1 = _agg_partials(y2, src, dst, zeros_p, n, p2, 1000)
    out16 = pl.pallas_call(
        _fin_body,
        grid=(g,),
        in_specs=[pl.BlockSpec((NC, bs, p2), lambda i: (0, i, 0)),
                  pl.BlockSpec((bs, p2), lambda i: (i, 0)),
                  pl.BlockSpec((bs, 1), lambda i: (i, 0)),
                  pl.BlockSpec((1, p2), lambda i: (0, 0))],
        out_specs=pl.BlockSpec((bs, p2), lambda i: (i, 0)),
        out_shape=_sds((n, p2)))(q1, y2, dinv, b2p)
    return out16[:, :out_dim]


# speedup vs baseline: 1.1554x; 1.0738x over previous
import functools

import jax
import jax.numpy as jnp
from jax import lax
from jax.experimental import pallas as pl
from jax.experimental.pallas import tpu as pltpu
from jax.experimental.pallas import tpu_sc as plsc

NC = 2
NS = 16
NW = NC * NS
CHUNK = 1000
DEGW = 8


def _mesh():
    return plsc.VectorSubcoreMesh(core_axis_name="c", subcore_axis_name="s")


def _deg_partials(dst, ones_hbm, zeros_hbm, n, chunk):
    e = dst.shape[0]
    nch = e // (NW * chunk)
    dst3 = dst.reshape(NW, nch, chunk)
    rows_per_out = n // 10

    @functools.partial(
        pl.kernel,
        out_type=jax.ShapeDtypeStruct((NC, n, DEGW), jnp.float32),
        mesh=_mesh(),
        scratch_types=[
            pltpu.VMEM((nch, chunk), jnp.int32),
            pltpu.VMEM((chunk, DEGW), jnp.float32),
            pltpu.VMEM_SHARED((n, DEGW), jnp.float32),
        ],
        compiler_params=pltpu.CompilerParams(use_tc_tiling_on_sc=False),
    )
    def deg_k(dst_hbm, ones_h, zeros_h, out_hbm, idx_d, ones_v, acc):
        ci = lax.axis_index("c")
        s = lax.axis_index("s")
        wid = ci * NS + s
        pltpu.sync_copy(dst_hbm.at[wid], idx_d)
        pltpu.sync_copy(ones_h, ones_v)

        @pl.when(s < 10)
        def _zero():
            sl = pl.ds(pl.multiple_of(s * rows_per_out, 8), rows_per_out)
            pltpu.sync_copy(zeros_h, acc.at[sl])

        plsc.subcore_barrier()

        def body(j, carry):
            pltpu.sync_copy(ones_v, acc.at[idx_d.at[j]], add=True)
            return carry

        lax.fori_loop(0, nch, body, 0)
        plsc.subcore_barrier()

        @pl.when(s < 10)
        def _out():
            sl = pl.ds(pl.multiple_of(s * rows_per_out, 8), rows_per_out)
            pltpu.sync_copy(acc.at[sl], out_hbm.at[ci].at[sl])

    return deg_k(dst3, ones_hbm, zeros_hbm)


def _agg_partials(y, src, dst, zeros_hbm, n, d, chunk):
    e = src.shape[0]
    nch = e // (NW * chunk)
    src3 = src.reshape(NW, nch, chunk)
    dst3 = dst.reshape(NW, nch, chunk)
    rows_per_out = n // 10

    @functools.partial(
        pl.kernel,
        out_type=jax.ShapeDtypeStruct((NC, n, 128), jnp.float32),
        mesh=_mesh(),
        scratch_types=[
            pltpu.VMEM((nch, chunk), jnp.int32),
            pltpu.VMEM((nch, chunk), jnp.int32),
            pltpu.VMEM((chunk, d), jnp.float32),
            pltpu.VMEM((chunk, d), jnp.float32),
            pltpu.VMEM_SHARED((n, d), jnp.float32),
            pltpu.SemaphoreType.DMA,
            pltpu.SemaphoreType.DMA,
            pltpu.SemaphoreType.DMA,
            pltpu.SemaphoreType.DMA,
        ],
        compiler_params=pltpu.CompilerParams(use_tc_tiling_on_sc=False),
    )
    def agg_k(y_hbm, src_hbm, dst_hbm, zeros_h, out_hbm,
              idx_s, idx_d, rows_a, rows_b, acc, ga, gb, sa, sb):
        ci = lax.axis_index("c")
        s = lax.axis_index("s")
        wid = ci * NS + s
        pltpu.sync_copy(src_hbm.at[wid], idx_s)
        pltpu.sync_copy(dst_hbm.at[wid], idx_d)

        @pl.when(s < 10)
        def _zero():
            sl = pl.ds(pl.multiple_of(s * rows_per_out, 8), rows_per_out)
            pltpu.sync_copy(zeros_h, acc.at[sl])

        plsc.subcore_barrier()

        def gather(j, buf, sem):
            return pltpu.async_copy(y_hbm.at[idx_s.at[j]], buf, sem)

        def scat(j, buf, sem):
            return pltpu.async_copy(buf, acc.at[idx_d.at[j]], sem, add=True)

        h_ga = gather(0, rows_a, ga)
        h_sb = None
        for i in range(nch // 2):
            h_ga.wait()
            if h_sb is not None:
                h_sb.wait()
            h_gb = gather(2 * i + 1, rows_b, gb)
            h_sa = scat(2 * i, rows_a, sa)
            h_gb.wait()
            h_sa.wait()
            if i + 1 < nch // 2:
                h_ga = gather(2 * i + 2, rows_a, ga)
            h_sb = scat(2 * i + 1, rows_b, sb)
        h_sb.wait()
        plsc.subcore_barrier()

        @pl.when(s < 10)
        def _out():
            sl = pl.ds(pl.multiple_of(s * rows_per_out, 8), rows_per_out)
            pltpu.sync_copy(acc.at[sl], out_hbm.at[ci].at[sl].at[:, pl.ds(0, d)])

    return agg_k(y, src3, dst3, zeros_hbm)


def _scale_body(x_ref, w1_ref, dpw_ref, y_ref, dinv_ref):
    n = x_ref.shape[0]
    cnt = dpw_ref[0] + dpw_ref[1]
    cnt = cnt.reshape(n // 16, 16, DEGW)[:, :, 0]
    deg = cnt.reshape(n, 1) + 1.0
    dinv = lax.rsqrt(deg)
    dinv_ref[...] = dinv
    z = jnp.dot(x_ref[...], w1_ref[...], preferred_element_type=jnp.float32)
    y_ref[...] = z * dinv


def _mid_body(p_ref, y1_ref, dinv_ref, b1_ref, w2_ref, y2_ref):
    hid = y1_ref.shape[1]
    agg = p_ref[0, :, :hid] + p_ref[1, :, :hid] + y1_ref[...]
    h = jnp.maximum(agg * dinv_ref[...] + b1_ref[...], 0.0)
    z2 = jnp.dot(h, w2_ref[...], preferred_element_type=jnp.float32)
    y2_ref[...] = z2 * dinv_ref[...]


def _fin_body(q_ref, y2_ref, dinv_ref, b2_ref, o_ref):
    p2 = y2_ref.shape[1]
    q = q_ref[0, :, :p2] + q_ref[1, :, :p2]
    o_ref[...] = (q + y2_ref[...]) * dinv_ref[...] + b2_ref[...]


def _sds(shape):
    return jax.ShapeDtypeStruct(shape, jnp.float32)


def kernel(x, edge_index, W1, b1, W2, b2):
    n, in_dim = x.shape
    hid = W1.shape[1]
    out_dim = W2.shape[1]
    p2 = 16
    src = edge_index[0].astype(jnp.int32)
    dst = edge_index[1].astype(jnp.int32)
    ones1 = jnp.ones((1000, DEGW), jnp.float32)
    zeros1 = jnp.zeros((n // 10, DEGW), jnp.float32)
    zeros_h = jnp.zeros((n // 10, hid), jnp.float32)
    zeros_p = jnp.zeros((n // 10, p2), jnp.float32)
    W2p = jnp.zeros((hid, p2), jnp.float32).at[:, :out_dim].set(W2)
    b1r = b1.reshape(1, hid)
    b2p = jnp.zeros((1, p2), jnp.float32).at[0, :out_dim].set(b2)

    bs = 2000
    g = n // bs

    degp = _deg_partials(dst, ones1, zeros1, n, 1000)
    dpw = degp.reshape(NC, n // (128 // DEGW), 128)

    y1, dinv = pl.pallas_call(
        _scale_body,
        out_shape=(_sds((n, hid)), _sds((n, 1))))(x, W1, dpw)

    p1 = _agg_partials(y1, src, dst, zeros_h, n, hid, 500)
    y2 = pl.pallas_call(
        _mid_body,
        grid=(g,),
        in_specs=[pl.BlockSpec((NC, bs, 128), lambda i: (0, i, 0)),
                  pl.BlockSpec((bs, hid), lambda i: (i, 0)),
                  pl.BlockSpec((bs, 1), lambda i: (i, 0)),
                  pl.BlockSpec((1, hid), lambda i: (0, 0)),
                  pl.BlockSpec((hid, p2), lambda i: (0, 0))],
        out_specs=pl.BlockSpec((bs, p2), lambda i: (i, 0)),
        out_shape=_sds((n, p2)))(p1, y1, dinv, b1r, W2p)

    q1 = _agg_partials(y2, src, dst, zeros_p, n, p2, 1000)
    out16 = pl.pallas_call(
        _fin_body,
        grid=(g,),
        in_specs=[pl.BlockSpec((NC, bs, 128), lambda i: (0, i, 0)),
                  pl.BlockSpec((bs, p2), lambda i: (i, 0)),
                  pl.BlockSpec((bs, 1), lambda i: (i, 0)),
                  pl.BlockSpec((1, p2), lambda i: (0, 0))],
        out_specs=pl.BlockSpec((bs, p2), lambda i: (i, 0)),
        out_shape=_sds((n, p2)))(q1, y2, dinv, b2p)
    return out16[:, :out_dim]


# speedup vs baseline: 1.1977x; 1.0366x over previous
import functools

import jax
import jax.numpy as jnp
from jax import lax
from jax.experimental import pallas as pl
from jax.experimental.pallas import tpu as pltpu
from jax.experimental.pallas import tpu_sc as plsc

NC = 2
NS = 16
NW = NC * NS
CHUNK = 1000
DEGW = 8


def _mesh():
    return plsc.VectorSubcoreMesh(core_axis_name="c", subcore_axis_name="s")


def _deg_partials(dst, ones_hbm, zeros_hbm, n, chunk):
    e = dst.shape[0]
    nch = e // (NW * chunk)
    dst3 = dst.reshape(NW, nch, chunk)
    rows_per_out = n // 10

    @functools.partial(
        pl.kernel,
        out_type=jax.ShapeDtypeStruct((NC, n, DEGW), jnp.float32),
        mesh=_mesh(),
        scratch_types=[
            pltpu.VMEM((nch, chunk), jnp.int32),
            pltpu.VMEM((chunk, DEGW), jnp.float32),
            pltpu.VMEM_SHARED((n, DEGW), jnp.float32),
        ],
        compiler_params=pltpu.CompilerParams(use_tc_tiling_on_sc=False),
    )
    def deg_k(dst_hbm, ones_h, zeros_h, out_hbm, idx_d, ones_v, acc):
        ci = lax.axis_index("c")
        s = lax.axis_index("s")
        wid = ci * NS + s
        pltpu.sync_copy(dst_hbm.at[wid], idx_d)
        pltpu.sync_copy(ones_h, ones_v)

        @pl.when(s < 10)
        def _zero():
            sl = pl.ds(pl.multiple_of(s * rows_per_out, 8), rows_per_out)
            pltpu.sync_copy(zeros_h, acc.at[sl])

        plsc.subcore_barrier()

        def body(j, carry):
            pltpu.sync_copy(ones_v, acc.at[idx_d.at[j]], add=True)
            return carry

        lax.fori_loop(0, nch, body, 0)
        plsc.subcore_barrier()

        @pl.when(s < 10)
        def _out():
            sl = pl.ds(pl.multiple_of(s * rows_per_out, 8), rows_per_out)
            pltpu.sync_copy(acc.at[sl], out_hbm.at[ci].at[sl])

    return deg_k(dst3, ones_hbm, zeros_hbm)


def _agg_partials(y, src, dst, zeros_hbm, n, d, chunk):
    e = src.shape[0]
    nch = e // (NW * chunk)
    src3 = src.reshape(NW, nch, chunk)
    dst3 = dst.reshape(NW, nch, chunk)
    rows_per_out = n // 10

    @functools.partial(
        pl.kernel,
        out_type=jax.ShapeDtypeStruct((NC, n, 128), jnp.float32),
        mesh=_mesh(),
        scratch_types=[
            pltpu.VMEM((nch, chunk), jnp.int32),
            pltpu.VMEM((nch, chunk), jnp.int32),
            [pltpu.VMEM((chunk, d), jnp.float32)] * 4,
            pltpu.VMEM_SHARED((n, d), jnp.float32),
            [pltpu.SemaphoreType.DMA] * 4,
            [pltpu.SemaphoreType.DMA] * 4,
        ],
        compiler_params=pltpu.CompilerParams(use_tc_tiling_on_sc=False),
    )
    def agg_k(y_hbm, src_hbm, dst_hbm, zeros_h, out_hbm,
              idx_s, idx_d, rows, acc, gs, ss):
        ci = lax.axis_index("c")
        s = lax.axis_index("s")
        wid = ci * NS + s
        pltpu.sync_copy(src_hbm.at[wid], idx_s)
        pltpu.sync_copy(dst_hbm.at[wid], idx_d)

        @pl.when(s < 10)
        def _zero():
            sl = pl.ds(pl.multiple_of(s * rows_per_out, 8), rows_per_out)
            pltpu.sync_copy(zeros_h, acc.at[sl])

        plsc.subcore_barrier()

        def gather(j, b):
            return pltpu.async_copy(y_hbm.at[idx_s.at[j]], rows[b], gs[b])

        def scat(j, b):
            return pltpu.async_copy(rows[b], acc.at[idx_d.at[j]], ss[b], add=True)

        gh = [None] * 4
        sh = [None] * 4
        gh[0] = gather(0, 0)
        if nch > 1:
            gh[1] = gather(1, 1)
        for i in range(nch):
            b = i % 4
            gh[i % 4].wait()
            sh[b] = scat(i, b)
            ni = i + 2
            if ni < nch:
                nb = ni % 4
                if sh[nb] is not None:
                    sh[nb].wait()
                    sh[nb] = None
                gh[nb] = gather(ni, nb)
        for b in range(4):
            if sh[b] is not None:
                sh[b].wait()
        plsc.subcore_barrier()

        @pl.when(s < 10)
        def _out():
            sl = pl.ds(pl.multiple_of(s * rows_per_out, 8), rows_per_out)
            pltpu.sync_copy(acc.at[sl], out_hbm.at[ci].at[sl].at[:, pl.ds(0, d)])

    return agg_k(y, src3, dst3, zeros_hbm)


def _scale_body(x_ref, w1_ref, dpw_ref, y_ref, dinv_ref):
    n = x_ref.shape[0]
    cnt = dpw_ref[0] + dpw_ref[1]
    cnt = cnt.reshape(n // 16, 16, DEGW)[:, :, 0]
    deg = cnt.reshape(n, 1) + 1.0
    dinv = lax.rsqrt(deg)
    dinv_ref[...] = dinv
    z = jnp.dot(x_ref[...], w1_ref[...], preferred_element_type=jnp.float32)
    y_ref[...] = z * dinv


def _mid_body(p_ref, y1_ref, dinv_ref, b1_ref, w2_ref, y2_ref):
    hid = y1_ref.shape[1]
    agg = p_ref[0, :, :hid] + p_ref[1, :, :hid] + y1_ref[...]
    h = jnp.maximum(agg * dinv_ref[...] + b1_ref[...], 0.0)
    z2 = jnp.dot(h, w2_ref[...], preferred_element_type=jnp.float32)
    y2_ref[...] = z2 * dinv_ref[...]


def _fin_body(q_ref, y2_ref, dinv_ref, b2_ref, o_ref):
    p2 = y2_ref.shape[1]
    q = q_ref[0, :, :p2] + q_ref[1, :, :p2]
    o_ref[...] = (q + y2_ref[...]) * dinv_ref[...] + b2_ref[...]


def _sds(shape):
    return jax.ShapeDtypeStruct(shape, jnp.float32)


def kernel(x, edge_index, W1, b1, W2, b2):
    n, in_dim = x.shape
    hid = W1.shape[1]
    out_dim = W2.shape[1]
    p2 = 16
    src = edge_index[0].astype(jnp.int32)
    dst = edge_index[1].astype(jnp.int32)
    ones1 = jnp.ones((1000, DEGW), jnp.float32)
    zeros1 = jnp.zeros((n // 10, DEGW), jnp.float32)
    zeros_h = jnp.zeros((n // 10, hid), jnp.float32)
    zeros_p = jnp.zeros((n // 10, p2), jnp.float32)
    W2p = jnp.zeros((hid, p2), jnp.float32).at[:, :out_dim].set(W2)
    b1r = b1.reshape(1, hid)
    b2p = jnp.zeros((1, p2), jnp.float32).at[0, :out_dim].set(b2)

    bs = 2000
    g = n // bs

    degp = _deg_partials(dst, ones1, zeros1, n, 1000)
    dpw = degp.reshape(NC, n // (128 // DEGW), 128)

    y1, dinv = pl.pallas_call(
        _scale_body,
        out_shape=(_sds((n, hid)), _sds((n, 1))))(x, W1, dpw)

    p1 = _agg_partials(y1, src, dst, zeros_h, n, hid, 250)
    y2 = pl.pallas_call(
        _mid_body,
        grid=(g,),
        in_specs=[pl.BlockSpec((NC, bs, 128), lambda i: (0, i, 0)),
                  pl.BlockSpec((bs, hid), lambda i: (i, 0)),
                  pl.BlockSpec((bs, 1), lambda i: (i, 0)),
                  pl.BlockSpec((1, hid), lambda i: (0, 0)),
                  pl.BlockSpec((hid, p2), lambda i: (0, 0))],
        out_specs=pl.BlockSpec((bs, p2), lambda i: (i, 0)),
        out_shape=_sds((n, p2)))(p1, y1, dinv, b1r, W2p)

    q1 = _agg_partials(y2, src, dst, zeros_p, n, p2, 1000)
    out16 = pl.pallas_call(
        _fin_body,
        grid=(g,),
        in_specs=[pl.BlockSpec((NC, bs, 128), lambda i: (0, i, 0)),
                  pl.BlockSpec((bs, p2), lambda i: (i, 0)),
                  pl.BlockSpec((bs, 1), lambda i: (i, 0)),
                  pl.BlockSpec((1, p2), lambda i: (0, 0))],
        out_specs=pl.BlockSpec((bs, p2), lambda i: (i, 0)),
        out_shape=_sds((n, p2)))(q1, y2, dinv, b2p)
    return out16[:, :out_dim]
